# baseline (speedup 1.0000x reference)
import functools

import jax
import jax.numpy as jnp
from jax import lax
from jax.experimental import pallas as pl
from jax.experimental.pallas import tpu as pltpu
from jax.experimental.pallas import tpu_sc as plsc

N = 16000
D = 512
NCH = 4
CW = 128
NPH = NCH // 2
E_LIST = (100000, 100000, 100000, 32000)
EPAD = (100352, 100352, 100352, 32768)
TYPE_BASE = (0, 100352, 200704, 301056)
EP4 = 333824
E_ALL = 350208
NB_ALL = 171
NT_ALL = E_ALL // 16

_f32 = jnp.float32
_i32 = jnp.int32


def _proj_body(x_ref, w_ref, b_ref, o_ref):
    acc = jnp.dot(x_ref[...], w_ref[...], preferred_element_type=_f32)
    o_ref[...] = acc + b_ref[...][None, :]


def _proj(x, w, b, bm):
    m, k = x.shape
    return pl.pallas_call(
        _proj_body,
        grid=(m // bm,),
        in_specs=[
            pl.BlockSpec((bm, k), lambda i: (i, 0)),
            pl.BlockSpec((k, D), lambda i: (0, 0)),
            pl.BlockSpec((D,), lambda i: (0,)),
        ],
        out_specs=pl.BlockSpec((bm, D), lambda i: (i, 0)),
        out_shape=jax.ShapeDtypeStruct((m, D), _f32),
    )(x, w, b)


def _layer_body_chunked(h_ref, w_ref, b_ref, o_ref):
    acc = jnp.dot(h_ref[0], w_ref[0], preferred_element_type=_f32)
    for c in range(1, NCH):
        acc += jnp.dot(h_ref[c], w_ref[c], preferred_element_type=_f32)
    acc = jnp.maximum(acc + b_ref[...][None, :], 0.0)
    for c in range(NCH):
        o_ref[c] = acc[:, c * CW:(c + 1) * CW]


def _layer_body_flat(h_ref, w_ref, b_ref, o_ref):
    acc = jnp.dot(h_ref[0], w_ref[0], preferred_element_type=_f32)
    for c in range(1, NCH):
        acc += jnp.dot(h_ref[c], w_ref[c], preferred_element_type=_f32)
    o_ref[...] = jnp.maximum(acc + b_ref[...][None, :], 0.0)


def _layer_tc(h_cm, w, b, chunked_out, bm=1000):
    h4 = h_cm.reshape(NCH, N, CW)
    w4 = w.reshape(NCH, CW, D)
    in_specs = [
        pl.BlockSpec((NCH, bm, CW), lambda i: (0, i, 0)),
        pl.BlockSpec((NCH, CW, D), lambda i: (0, 0, 0)),
        pl.BlockSpec((D,), lambda i: (0,)),
    ]
    if chunked_out:
        out = pl.pallas_call(
            _layer_body_chunked,
            grid=(N // bm,),
            in_specs=in_specs,
            out_specs=pl.BlockSpec((NCH, bm, CW), lambda i: (0, i, 0)),
            out_shape=jax.ShapeDtypeStruct((NCH, N, CW), _f32),
        )(h4, w4, b)
        return out.reshape(NCH * N, CW)
    return pl.pallas_call(
        _layer_body_flat,
        grid=(N // bm,),
        in_specs=in_specs,
        out_specs=pl.BlockSpec((bm, D), lambda i: (i, 0)),
        out_shape=jax.ShapeDtypeStruct((N, D), _f32),
    )(h4, w4, b)


_MESH = plsc.VectorSubcoreMesh(core_axis_name="c", subcore_axis_name="s")


@functools.partial(
    pl.kernel,
    out_type=(
        jax.ShapeDtypeStruct((EP4,), _f32),
        jax.ShapeDtypeStruct((EP4,), _f32),
        jax.ShapeDtypeStruct((2 * N,), _f32),
    ),
    mesh=_MESH,
    scratch_types=[
        pltpu.VMEM_SHARED((N,), _f32),
        pltpu.VMEM((1008,), _f32),
        pltpu.VMEM((128,), _i32),
        pltpu.VMEM((128,), _f32),
        pltpu.VMEM((128,), _f32),
        pltpu.VMEM((128,), _f32),
        pltpu.VMEM((128,), _f32),
        pltpu.VMEM((1024,), _f32),
        pltpu.VMEM((16,), _f32),
        pltpu.SemaphoreType.DMA,
    ],
)
def _prep_sc(dst_hbm, val_hbm, betas_hbm, sv0_hbm, sv1_hbm, dinv_hbm,
             deg_sh, dslice, dstb, valb, dvb, o0, o1, zbuf, btile, sem):
    cid = lax.axis_index("c")
    wid = lax.axis_index("s")

    def _zb(i, c):
        zbuf[pl.ds(16 * i, 16)] = jnp.zeros((16,), _f32)
        return c
    lax.fori_loop(0, 64, _zb, 0)
    pltpu.sync_copy(betas_hbm, btile)

    for t in range(4):
        nt = EPAD[t] // 16
        nb = nt // 128
        base = TYPE_BASE[t]

        @pl.when(cid == (t % 2))
        def _type_block():
            pltpu.sync_copy(zbuf.at[pl.ds(0, 1000)],
                            deg_sh.at[pl.ds(wid * 1000, 1000)])
            plsc.subcore_barrier()

            def _deg(g, c):
                off = base + wid * nt + g * 128
                pltpu.sync_copy(dst_hbm.at[pl.ds(off, 128)], dstb)
                pltpu.sync_copy(val_hbm.at[pl.ds(off, 128)], valb)
                pltpu.sync_copy(valb, deg_sh.at[dstb], add=True)
                return c
            lax.fori_loop(0, nb, _deg, 0)
            plsc.subcore_barrier()

            pltpu.sync_copy(deg_sh.at[pl.ds(wid * 1000, 1000)],
                            dslice.at[pl.ds(0, 1000)])

            def _inv(i, c):
                v = dslice[pl.ds(16 * i, 16)]
                pos = v > 0.0
                dslice[pl.ds(16 * i, 16)] = jnp.where(
                    pos, 1.0 / jnp.where(pos, v, 1.0), 0.0)
                return c
            lax.fori_loop(0, 63, _inv, 0)
            pltpu.sync_copy(dslice.at[pl.ds(0, 1000)],
                            dinv_hbm.at[pl.ds(cid * N + wid * 1000, 1000)])
            plsc.subcore_barrier()

            bvec = btile[pl.ds(0, 16)]
            b0s = bvec[t]
            b1s = bvec[8 + t]
            tab_off = cid * N

            def _sval(g, c):
                off = base + wid * nt + g * 128
                pltpu.sync_copy(dst_hbm.at[pl.ds(off, 128)], dstb)
                pltpu.sync_copy(val_hbm.at[pl.ds(off, 128)], valb)
                for j in range(8):
                    sl = pl.ds(16 * j, 16)
                    dstb[sl] = dstb[sl] + tab_off
                pltpu.async_copy(dinv_hbm.at[dstb], dvb, sem).wait()
                for j in range(8):
                    sl = pl.ds(16 * j, 16)
                    nv = valb[sl] * dvb[sl]
                    o0[sl] = nv * b0s
                    o1[sl] = nv * b1s
                pltpu.sync_copy(o0, sv0_hbm.at[pl.ds(off, 128)])
                pltpu.sync_copy(o1, sv1_hbm.at[pl.ds(off, 128)])
                return c
            lax.fori_loop(0, nb, _sval, 0)


NH = 8000
NDUMP = 64


@functools.partial(
    pl.kernel,
    out_type=jax.ShapeDtypeStruct((NCH * N, CW), _f32),
    mesh=_MESH,
    scratch_types=[
        pltpu.VMEM_SHARED((NH + NDUMP, CW), _f32),
        pltpu.VMEM((128,), _i32),
        pltpu.VMEM((128,), _i32),
        pltpu.VMEM((128,), _f32),
        pltpu.VMEM((128, CW), _f32),
        pltpu.SemaphoreType.DMA,
    ],
)
def _spmm_sc(src_hbm, dst_hbm, sval_hbm, x_hbm, out_hbm,
             acc, idxb, dstb, valb, rows, sem):
    cid = lax.axis_index("c")
    wid = lax.axis_index("s")

    def _phase(p, carry):
        chunk = 2 * (p // 2) + cid
        coff = chunk * N
        hoff = (p % 2) * NH

        def _zr(i, c):
            for j in range(CW // 16):
                rows[i, pl.ds(16 * j, 16)] = jnp.zeros((16,), _f32)
            return c
        lax.fori_loop(0, 128, _zr, 0)
        for i in range(3):
            pltpu.sync_copy(rows, acc.at[pl.ds(wid * 504 + i * 128, 128)])
        pltpu.sync_copy(rows.at[pl.ds(0, 120)],
                        acc.at[pl.ds(wid * 504 + 384, 120)])
        plsc.subcore_barrier()

        def _edge(g, c):
            off = wid * NT_ALL + g * 128
            pltpu.sync_copy(src_hbm.at[pl.ds(off, 128)], idxb)
            pltpu.sync_copy(dst_hbm.at[pl.ds(off, 128)], dstb)
            pltpu.sync_copy(sval_hbm.at[pl.ds(off, 128)], valb)
            for k in range(8):
                sl = pl.ds(16 * k, 16)
                gv = idxb[sl] + coff
                pltpu.async_copy(x_hbm.at[gv], rows.at[sl], sem).wait()

            def _scale(j16, cc):
                vv = valb[pl.ds(16 * j16, 16)]
                for lane in range(16):
                    s = vv[lane]
                    r = j16 * 16 + lane
                    for j in range(CW // 16):
                        sl = pl.ds(16 * j, 16)
                        rows[r, sl] = rows[r, sl] * s
                return cc
            lax.fori_loop(0, 8, _scale, 0)
            for k in range(8):
                sl = pl.ds(16 * k, 16)
                dv = dstb[sl] - hoff
                msk = (dv >= 0) & (dv < NH)
                dump = NH + ((lax.iota(_i32, 16) + 16 * k) & (NDUMP - 1))
                iv = jnp.where(msk, dv, dump)
                pltpu.sync_copy(rows.at[sl], acc.at[iv], add=True)
            return c
        lax.fori_loop(0, NB_ALL, _edge, 0)
        plsc.subcore_barrier()

        @pl.when(wid < 8)
        def _ro():
            for i in range(5):
                pltpu.sync_copy(
                    acc.at[pl.ds(wid * 1000 + i * 200, 200)],
                    out_hbm.at[pl.ds(coff + hoff + wid * 1000 + i * 200, 200)])
        plsc.subcore_barrier()
        return carry

    lax.fori_loop(0, 4, _phase, 0)


def _pad_idx(npad, salt):
    return ((jnp.arange(npad, dtype=_i32) * 131) + salt) % N


def kernel(x0, x1, x2, x3, x4,
           edge_index_0, edge_index_1, edge_index_2, edge_index_3,
           edge_value_0, edge_value_1, edge_value_2, edge_value_3,
           Wp0, Wp1, Wp2, Wp3, Wp4,
           bp0, bp1, bp2, bp3, bp4,
           alpha0, alpha1, W0, b0, W1, b1):
    xs = (x0, x1, x2, x3, x4)
    Wps = (Wp0, Wp1, Wp2, Wp3, Wp4)
    bps = (bp0, bp1, bp2, bp3, bp4)
    eidx = (edge_index_0, edge_index_1, edge_index_2, edge_index_3)
    evals = (edge_value_0, edge_value_1, edge_value_2, edge_value_3)

    beta_l0 = jnp.mean(jax.nn.softmax(alpha0, axis=-1), axis=0)
    beta_l1 = jnp.mean(jax.nn.softmax(alpha1, axis=-1), axis=0)
    betas16 = jnp.zeros((16,), _f32)
    betas16 = betas16.at[0:4].set(beta_l0[0:4]).at[8:12].set(beta_l1[0:4])

    X = jnp.concatenate(
        [_proj(x, w, b, bm=1000) for x, w, b in zip(xs, Wps, bps)], axis=0)
    x_cm = X.reshape(N, NCH, CW).transpose(1, 0, 2).reshape(NCH * N, CW)

    srcs, dsts, vals = [], [], []
    for t in range(4):
        npad = EPAD[t] - E_LIST[t]
        srcs += [eidx[t][0], _pad_idx(npad, 7 * t + 1)]
        dsts += [eidx[t][1], _pad_idx(npad, 13 * t + 3)]
        vals += [evals[t], jnp.zeros((npad,), _f32)]
    self_idx = jnp.arange(N, dtype=_i32)
    tail = E_ALL - EP4 - N
    srcs += [self_idx, _pad_idx(tail, 5)]
    dsts += [self_idx, _pad_idx(tail, 9)]
    vals += [jnp.zeros((N + tail,), _f32)]
    src_all = jnp.concatenate(srcs)
    dst_all = jnp.concatenate(dsts)
    val_all = jnp.concatenate(vals)

    sv0_p, sv1_p, _ = _prep_sc(dst_all, val_all, betas16)
    sval0 = jnp.concatenate(
        [sv0_p, jnp.full((N,), beta_l0[4], _f32), jnp.zeros((tail,), _f32)])
    sval1 = jnp.concatenate(
        [sv1_p, jnp.full((N,), beta_l1[4], _f32), jnp.zeros((tail,), _f32)])

    h1 = _spmm_sc(src_all, dst_all, sval0, x_cm)
    x2_cm = _layer_tc(h1, W0, b0, chunked_out=True)

    h2 = _spmm_sc(src_all, dst_all, sval1, x2_cm)
    return _layer_tc(h2, W1, b1, chunked_out=False)


# speedup vs baseline: 3.6735x; 3.6735x over previous
import functools

import jax
import jax.numpy as jnp
from jax import lax
from jax.experimental import pallas as pl
from jax.experimental.pallas import tpu as pltpu
from jax.experimental.pallas import tpu_sc as plsc

N = 16000
D = 512
NCH = 4
CW = 128
NPH = NCH // 2
E_LIST = (100000, 100000, 100000, 32000)
EPAD = (100352, 100352, 100352, 32768)
TYPE_BASE = (0, 100352, 200704, 301056)
EP4 = 333824
E_ALL = 350208
NB_ALL = 171
NT_ALL = E_ALL // 16

_f32 = jnp.float32
_i32 = jnp.int32


def _proj_body(x_ref, w_ref, b_ref, o_ref):
    acc = jnp.dot(x_ref[...], w_ref[...], preferred_element_type=_f32)
    o_ref[...] = acc + b_ref[...][None, :]


def _proj(x, w, b, bm):
    m, k = x.shape
    return pl.pallas_call(
        _proj_body,
        grid=(m // bm,),
        in_specs=[
            pl.BlockSpec((bm, k), lambda i: (i, 0)),
            pl.BlockSpec((k, D), lambda i: (0, 0)),
            pl.BlockSpec((D,), lambda i: (0,)),
        ],
        out_specs=pl.BlockSpec((bm, D), lambda i: (i, 0)),
        out_shape=jax.ShapeDtypeStruct((m, D), _f32),
    )(x, w, b)


def _layer_body_chunked(h_ref, w_ref, b_ref, o_ref):
    acc = jnp.dot(h_ref[0], w_ref[0], preferred_element_type=_f32)
    for c in range(1, NCH):
        acc += jnp.dot(h_ref[c], w_ref[c], preferred_element_type=_f32)
    acc = jnp.maximum(acc + b_ref[...][None, :], 0.0)
    for c in range(NCH):
        o_ref[c] = acc[:, c * CW:(c + 1) * CW]


def _layer_body_flat(h_ref, w_ref, b_ref, o_ref):
    acc = jnp.dot(h_ref[0], w_ref[0], preferred_element_type=_f32)
    for c in range(1, NCH):
        acc += jnp.dot(h_ref[c], w_ref[c], preferred_element_type=_f32)
    o_ref[...] = jnp.maximum(acc + b_ref[...][None, :], 0.0)


def _layer_tc(h_cm, w, b, chunked_out, bm=1000):
    h4 = h_cm.reshape(NCH, N, CW)
    w4 = w.reshape(NCH, CW, D)
    in_specs = [
        pl.BlockSpec((NCH, bm, CW), lambda i: (0, i, 0)),
        pl.BlockSpec((NCH, CW, D), lambda i: (0, 0, 0)),
        pl.BlockSpec((D,), lambda i: (0,)),
    ]
    if chunked_out:
        out = pl.pallas_call(
            _layer_body_chunked,
            grid=(N // bm,),
            in_specs=in_specs,
            out_specs=pl.BlockSpec((NCH, bm, CW), lambda i: (0, i, 0)),
            out_shape=jax.ShapeDtypeStruct((NCH, N, CW), _f32),
        )(h4, w4, b)
        return out.reshape(NCH * N, CW)
    return pl.pallas_call(
        _layer_body_flat,
        grid=(N // bm,),
        in_specs=in_specs,
        out_specs=pl.BlockSpec((bm, D), lambda i: (i, 0)),
        out_shape=jax.ShapeDtypeStruct((N, D), _f32),
    )(h4, w4, b)


_MESH = plsc.VectorSubcoreMesh(core_axis_name="c", subcore_axis_name="s")


@functools.partial(
    pl.kernel,
    out_type=(
        jax.ShapeDtypeStruct((EP4,), _f32),
        jax.ShapeDtypeStruct((EP4,), _f32),
        jax.ShapeDtypeStruct((2 * N,), _f32),
    ),
    mesh=_MESH,
    scratch_types=[
        pltpu.VMEM_SHARED((N,), _f32),
        pltpu.VMEM((1008,), _f32),
        pltpu.VMEM((128,), _i32),
        pltpu.VMEM((128,), _f32),
        pltpu.VMEM((128,), _f32),
        pltpu.VMEM((128,), _f32),
        pltpu.VMEM((128,), _f32),
        pltpu.VMEM((1024,), _f32),
        pltpu.VMEM((16,), _f32),
        pltpu.SemaphoreType.DMA,
    ],
)
def _prep_sc(dst_hbm, val_hbm, betas_hbm, sv0_hbm, sv1_hbm, dinv_hbm,
             deg_sh, dslice, dstb, valb, dvb, o0, o1, zbuf, btile, sem):
    cid = lax.axis_index("c")
    wid = lax.axis_index("s")

    def _zb(i, c):
        zbuf[pl.ds(16 * i, 16)] = jnp.zeros((16,), _f32)
        return c
    lax.fori_loop(0, 64, _zb, 0)
    pltpu.sync_copy(betas_hbm, btile)

    for t in range(4):
        nt = EPAD[t] // 16
        nb = nt // 128
        base = TYPE_BASE[t]

        @pl.when(cid == (t % 2))
        def _type_block():
            pltpu.sync_copy(zbuf.at[pl.ds(0, 1000)],
                            deg_sh.at[pl.ds(wid * 1000, 1000)])
            plsc.subcore_barrier()

            def _deg(g, c):
                off = base + wid * nt + g * 128
                pltpu.sync_copy(dst_hbm.at[pl.ds(off, 128)], dstb)
                pltpu.sync_copy(val_hbm.at[pl.ds(off, 128)], valb)
                pltpu.sync_copy(valb, deg_sh.at[dstb], add=True)
                return c
            lax.fori_loop(0, nb, _deg, 0)
            plsc.subcore_barrier()

            pltpu.sync_copy(deg_sh.at[pl.ds(wid * 1000, 1000)],
                            dslice.at[pl.ds(0, 1000)])

            def _inv(i, c):
                v = dslice[pl.ds(16 * i, 16)]
                pos = v > 0.0
                dslice[pl.ds(16 * i, 16)] = jnp.where(
                    pos, 1.0 / jnp.where(pos, v, 1.0), 0.0)
                return c
            lax.fori_loop(0, 63, _inv, 0)
            pltpu.sync_copy(dslice.at[pl.ds(0, 1000)],
                            dinv_hbm.at[pl.ds(cid * N + wid * 1000, 1000)])
            plsc.subcore_barrier()

            bvec = btile[pl.ds(0, 16)]
            b0s = bvec[t]
            b1s = bvec[8 + t]
            tab_off = cid * N

            def _sval(g, c):
                off = base + wid * nt + g * 128
                pltpu.sync_copy(dst_hbm.at[pl.ds(off, 128)], dstb)
                pltpu.sync_copy(val_hbm.at[pl.ds(off, 128)], valb)
                for j in range(8):
                    sl = pl.ds(16 * j, 16)
                    dstb[sl] = dstb[sl] + tab_off
                pltpu.async_copy(dinv_hbm.at[dstb], dvb, sem).wait()
                for j in range(8):
                    sl = pl.ds(16 * j, 16)
                    nv = valb[sl] * dvb[sl]
                    o0[sl] = nv * b0s
                    o1[sl] = nv * b1s
                pltpu.sync_copy(o0, sv0_hbm.at[pl.ds(off, 128)])
                pltpu.sync_copy(o1, sv1_hbm.at[pl.ds(off, 128)])
                return c
            lax.fori_loop(0, nb, _sval, 0)


NH = 8000
NDUMP = 64


BE = 64
NBB = NT_ALL // BE


@functools.partial(
    pl.kernel,
    out_type=jax.ShapeDtypeStruct((NCH * N, CW), _f32),
    mesh=_MESH,
    scratch_types=[
        pltpu.VMEM_SHARED((NH + NDUMP, CW), _f32),
        pltpu.VMEM((NT_ALL,), _i32),
        pltpu.VMEM((NT_ALL,), _f32),
        pltpu.VMEM((BE, CW), _f32),
        pltpu.VMEM((BE, CW), _f32),
        pltpu.SemaphoreType.DMA,
        pltpu.SemaphoreType.DMA,
    ],
)
def _spmm_sc(pck_hbm, sval_hbm, x_hbm, out_hbm,
             acc, pckb, valb, rows0, rows1, sem0, sem1):
    cid = lax.axis_index("c")
    wid = lax.axis_index("s")
    rbufs = (rows0, rows1)
    sems = (sem0, sem1)

    pltpu.sync_copy(pck_hbm.at[pl.ds(wid * NT_ALL, NT_ALL)], pckb)
    pltpu.sync_copy(sval_hbm.at[pl.ds(wid * NT_ALL, NT_ALL)], valb)

    def _fire(g, b, coff):
        for k in range(4):
            pv = pckb[pl.ds(g * BE + 16 * k, 16)]
            gv = (pv & 0xFFFF) + coff
            pltpu.async_copy(x_hbm.at[gv], rbufs[b].at[pl.ds(16 * k, 16)],
                             sems[b])

    def _drain(b):
        pltpu.make_async_copy(x_hbm.at[pl.ds(0, BE)], rbufs[b],
                              sems[b]).wait()

    def _process(g, b, hoff):
        rows = rbufs[b]

        def _scale(j16, cc):
            vv = valb[pl.ds(g * BE + 16 * j16, 16)]
            for lane in range(16):
                s = vv[lane]
                r = j16 * 16 + lane
                for j in range(CW // 16):
                    sl = pl.ds(16 * j, 16)
                    rows[r, sl] = rows[r, sl] * s
            return cc
        lax.fori_loop(0, BE // 16, _scale, 0)
        for k in range(4):
            pv = pckb[pl.ds(g * BE + 16 * k, 16)]
            dv = lax.shift_right_logical(pv, 16) - hoff
            msk = (dv >= 0) & (dv < NH)
            dump = NH + ((lax.iota(_i32, 16) + 16 * k) & (NDUMP - 1))
            iv = jnp.where(msk, dv, dump)
            pltpu.sync_copy(rows.at[pl.ds(16 * k, 16)], acc.at[iv], add=True)

    def _phase(p, carry):
        chunk = 2 * (p // 2) + cid
        coff = chunk * N
        hoff = (p % 2) * NH

        def _zr(i, c):
            for j in range(CW // 16):
                rows0[i, pl.ds(16 * j, 16)] = jnp.zeros((16,), _f32)
            return c
        lax.fori_loop(0, BE, _zr, 0)
        for i in range(7):
            pltpu.sync_copy(rows0, acc.at[pl.ds(wid * 504 + i * BE, BE)])
        pltpu.sync_copy(rows0.at[pl.ds(0, 56)],
                        acc.at[pl.ds(wid * 504 + 448, 56)])
        plsc.subcore_barrier()

        _fire(0, 0, coff)

        def _pair(g2, c):
            g = g2 * 2
            _drain(0)
            _fire(g + 1, 1, coff)
            _process(g, 0, hoff)
            _drain(1)
            _fire(g + 2, 0, coff)
            _process(g + 1, 1, hoff)
            return c
        lax.fori_loop(0, NBB // 2 - 1, _pair, 0)
        _drain(0)
        _fire(NBB - 1, 1, coff)
        _process(NBB - 2, 0, hoff)
        _drain(1)
        _process(NBB - 1, 1, hoff)
        plsc.subcore_barrier()

        @pl.when(wid < 8)
        def _ro():
            for i in range(5):
                pltpu.sync_copy(
                    acc.at[pl.ds(wid * 1000 + i * 200, 200)],
                    out_hbm.at[pl.ds(coff + hoff + wid * 1000 + i * 200, 200)])
        plsc.subcore_barrier()
        return carry

    lax.fori_loop(0, 4, _phase, 0)


def _pad_idx(npad, salt):
    return ((jnp.arange(npad, dtype=_i32) * 131) + salt) % N


def kernel(x0, x1, x2, x3, x4,
           edge_index_0, edge_index_1, edge_index_2, edge_index_3,
           edge_value_0, edge_value_1, edge_value_2, edge_value_3,
           Wp0, Wp1, Wp2, Wp3, Wp4,
           bp0, bp1, bp2, bp3, bp4,
           alpha0, alpha1, W0, b0, W1, b1):
    xs = (x0, x1, x2, x3, x4)
    Wps = (Wp0, Wp1, Wp2, Wp3, Wp4)
    bps = (bp0, bp1, bp2, bp3, bp4)
    eidx = (edge_index_0, edge_index_1, edge_index_2, edge_index_3)
    evals = (edge_value_0, edge_value_1, edge_value_2, edge_value_3)

    beta_l0 = jnp.mean(jax.nn.softmax(alpha0, axis=-1), axis=0)
    beta_l1 = jnp.mean(jax.nn.softmax(alpha1, axis=-1), axis=0)
    betas16 = jnp.zeros((16,), _f32)
    betas16 = betas16.at[0:4].set(beta_l0[0:4]).at[8:12].set(beta_l1[0:4])

    X = jnp.concatenate(
        [_proj(x, w, b, bm=1000) for x, w, b in zip(xs, Wps, bps)], axis=0)
    x_cm = X.reshape(N, NCH, CW).transpose(1, 0, 2).reshape(NCH * N, CW)

    srcs, dsts, vals = [], [], []
    for t in range(4):
        npad = EPAD[t] - E_LIST[t]
        srcs += [eidx[t][0], _pad_idx(npad, 7 * t + 1)]
        dsts += [eidx[t][1], _pad_idx(npad, 13 * t + 3)]
        vals += [evals[t], jnp.zeros((npad,), _f32)]
    self_idx = jnp.arange(N, dtype=_i32)
    tail = E_ALL - EP4 - N
    srcs += [self_idx, _pad_idx(tail, 5)]
    dsts += [self_idx, _pad_idx(tail, 9)]
    vals += [jnp.zeros((N + tail,), _f32)]
    src_all = jnp.concatenate(srcs)
    dst_all = jnp.concatenate(dsts)
    val_all = jnp.concatenate(vals)
    pck_all = src_all | (dst_all << 16)

    sv0_p, sv1_p, _ = _prep_sc(dst_all, val_all, betas16)
    sval0 = jnp.concatenate(
        [sv0_p, jnp.full((N,), beta_l0[4], _f32), jnp.zeros((tail,), _f32)])
    sval1 = jnp.concatenate(
        [sv1_p, jnp.full((N,), beta_l1[4], _f32), jnp.zeros((tail,), _f32)])

    h1 = _spmm_sc(pck_all, sval0, x_cm)
    x2_cm = _layer_tc(h1, W0, b0, chunked_out=True)

    h2 = _spmm_sc(pck_all, sval1, x2_cm)
    return _layer_tc(h2, W1, b1, chunked_out=False)


# speedup vs baseline: 3.9546x; 1.0765x over previous
import functools

import jax
import jax.numpy as jnp
from jax import lax
from jax.experimental import pallas as pl
from jax.experimental.pallas import tpu as pltpu
from jax.experimental.pallas import tpu_sc as plsc

N = 16000
D = 512
NCH = 4
CW = 128
NPH = NCH // 2
E_LIST = (100000, 100000, 100000, 32000)
EPAD = (100352, 100352, 100352, 32768)
TYPE_BASE = (0, 100352, 200704, 301056)
EP4 = 333824
E_ALL = 350208
NB_ALL = 171
NT_ALL = E_ALL // 16

_f32 = jnp.float32
_i32 = jnp.int32


def _proj_body(x_ref, w_ref, b_ref, o_ref):
    acc = jnp.dot(x_ref[...], w_ref[...], preferred_element_type=_f32)
    o_ref[...] = acc + b_ref[...][None, :]


def _proj(x, w, b, bm):
    m, k = x.shape
    return pl.pallas_call(
        _proj_body,
        grid=(m // bm,),
        in_specs=[
            pl.BlockSpec((bm, k), lambda i: (i, 0)),
            pl.BlockSpec((k, D), lambda i: (0, 0)),
            pl.BlockSpec((D,), lambda i: (0,)),
        ],
        out_specs=pl.BlockSpec((bm, D), lambda i: (i, 0)),
        out_shape=jax.ShapeDtypeStruct((m, D), _f32),
    )(x, w, b)


def _layer_body_chunked(h_ref, w_ref, b_ref, o_ref):
    acc = jnp.dot(h_ref[0], w_ref[0], preferred_element_type=_f32)
    for c in range(1, NCH):
        acc += jnp.dot(h_ref[c], w_ref[c], preferred_element_type=_f32)
    acc = jnp.maximum(acc + b_ref[...][None, :], 0.0)
    for c in range(NCH):
        o_ref[c] = acc[:, c * CW:(c + 1) * CW]


def _layer_body_flat(h_ref, w_ref, b_ref, o_ref):
    acc = jnp.dot(h_ref[0], w_ref[0], preferred_element_type=_f32)
    for c in range(1, NCH):
        acc += jnp.dot(h_ref[c], w_ref[c], preferred_element_type=_f32)
    o_ref[...] = jnp.maximum(acc + b_ref[...][None, :], 0.0)


def _layer_tc(h_cm, w, b, chunked_out, bm=1000):
    h4 = h_cm.reshape(NCH, N, CW)
    w4 = w.reshape(NCH, CW, D)
    in_specs = [
        pl.BlockSpec((NCH, bm, CW), lambda i: (0, i, 0)),
        pl.BlockSpec((NCH, CW, D), lambda i: (0, 0, 0)),
        pl.BlockSpec((D,), lambda i: (0,)),
    ]
    if chunked_out:
        out = pl.pallas_call(
            _layer_body_chunked,
            grid=(N // bm,),
            in_specs=in_specs,
            out_specs=pl.BlockSpec((NCH, bm, CW), lambda i: (0, i, 0)),
            out_shape=jax.ShapeDtypeStruct((NCH, N, CW), _f32),
        )(h4, w4, b)
        return out.reshape(NCH * N, CW)
    return pl.pallas_call(
        _layer_body_flat,
        grid=(N // bm,),
        in_specs=in_specs,
        out_specs=pl.BlockSpec((bm, D), lambda i: (i, 0)),
        out_shape=jax.ShapeDtypeStruct((N, D), _f32),
    )(h4, w4, b)


_MESH = plsc.VectorSubcoreMesh(core_axis_name="c", subcore_axis_name="s")


@functools.partial(
    pl.kernel,
    out_type=(
        jax.ShapeDtypeStruct((EP4,), _f32),
        jax.ShapeDtypeStruct((EP4,), _f32),
        jax.ShapeDtypeStruct((2 * N,), _f32),
    ),
    mesh=_MESH,
    scratch_types=[
        pltpu.VMEM_SHARED((N,), _f32),
        pltpu.VMEM((1008,), _f32),
        pltpu.VMEM((128,), _i32),
        pltpu.VMEM((128,), _f32),
        pltpu.VMEM((128,), _f32),
        pltpu.VMEM((128,), _f32),
        pltpu.VMEM((128,), _f32),
        pltpu.VMEM((1024,), _f32),
        pltpu.VMEM((16,), _f32),
        pltpu.SemaphoreType.DMA,
    ],
)
def _prep_sc(dst_hbm, val_hbm, betas_hbm, sv0_hbm, sv1_hbm, dinv_hbm,
             deg_sh, dslice, dstb, valb, dvb, o0, o1, zbuf, btile, sem):
    cid = lax.axis_index("c")
    wid = lax.axis_index("s")

    def _zb(i, c):
        zbuf[pl.ds(16 * i, 16)] = jnp.zeros((16,), _f32)
        return c
    lax.fori_loop(0, 64, _zb, 0)
    pltpu.sync_copy(betas_hbm, btile)

    for t in range(4):
        nt = EPAD[t] // 16
        nb = nt // 128
        base = TYPE_BASE[t]

        @pl.when(cid == (t % 2))
        def _type_block():
            pltpu.sync_copy(zbuf.at[pl.ds(0, 1000)],
                            deg_sh.at[pl.ds(wid * 1000, 1000)])
            plsc.subcore_barrier()

            def _deg(g, c):
                off = base + wid * nt + g * 128
                pltpu.sync_copy(dst_hbm.at[pl.ds(off, 128)], dstb)
                pltpu.sync_copy(val_hbm.at[pl.ds(off, 128)], valb)
                pltpu.sync_copy(valb, deg_sh.at[dstb], add=True)
                return c
            lax.fori_loop(0, nb, _deg, 0)
            plsc.subcore_barrier()

            pltpu.sync_copy(deg_sh.at[pl.ds(wid * 1000, 1000)],
                            dslice.at[pl.ds(0, 1000)])

            def _inv(i, c):
                v = dslice[pl.ds(16 * i, 16)]
                pos = v > 0.0
                dslice[pl.ds(16 * i, 16)] = jnp.where(
                    pos, 1.0 / jnp.where(pos, v, 1.0), 0.0)
                return c
            lax.fori_loop(0, 63, _inv, 0)
            pltpu.sync_copy(dslice.at[pl.ds(0, 1000)],
                            dinv_hbm.at[pl.ds(cid * N + wid * 1000, 1000)])
            plsc.subcore_barrier()

            bvec = btile[pl.ds(0, 16)]
            b0s = bvec[t]
            b1s = bvec[8 + t]
            tab_off = cid * N

            def _sval(g, c):
                off = base + wid * nt + g * 128
                pltpu.sync_copy(dst_hbm.at[pl.ds(off, 128)], dstb)
                pltpu.sync_copy(val_hbm.at[pl.ds(off, 128)], valb)
                for j in range(8):
                    sl = pl.ds(16 * j, 16)
                    dstb[sl] = dstb[sl] + tab_off
                pltpu.async_copy(dinv_hbm.at[dstb], dvb, sem).wait()
                for j in range(8):
                    sl = pl.ds(16 * j, 16)
                    nv = valb[sl] * dvb[sl]
                    o0[sl] = nv * b0s
                    o1[sl] = nv * b1s
                pltpu.sync_copy(o0, sv0_hbm.at[pl.ds(off, 128)])
                pltpu.sync_copy(o1, sv1_hbm.at[pl.ds(off, 128)])
                return c
            lax.fori_loop(0, nb, _sval, 0)


NH = 8000
NDUMP = 64


BE = 64
NBB = NT_ALL // BE


@functools.partial(
    pl.kernel,
    out_type=jax.ShapeDtypeStruct((NCH * N, CW), _f32),
    mesh=_MESH,
    scratch_types=[
        pltpu.VMEM_SHARED((NH + NDUMP, CW), _f32),
        pltpu.VMEM((NT_ALL,), _i32),
        pltpu.VMEM((NT_ALL,), _f32),
        pltpu.VMEM((BE, CW), _f32),
        pltpu.VMEM((BE, CW), _f32),
        pltpu.VMEM((BE,), _i32),
        pltpu.VMEM((BE,), _i32),
        pltpu.SemaphoreType.DMA,
        pltpu.SemaphoreType.DMA,
        pltpu.SemaphoreType.DMA,
        pltpu.SemaphoreType.DMA,
    ],
)
def _spmm_sc(pck_hbm, sval_hbm, x_hbm, out_hbm,
             acc, pckb, valb, rows0, rows1, iv0, iv1,
             sem0, sem1, ssem0, ssem1):
    cid = lax.axis_index("c")
    wid = lax.axis_index("s")
    rbufs = (rows0, rows1)
    ivbufs = (iv0, iv1)
    sems = (sem0, sem1)
    ssems = (ssem0, ssem1)

    pltpu.sync_copy(pck_hbm.at[pl.ds(wid * NT_ALL, NT_ALL)], pckb)
    pltpu.sync_copy(sval_hbm.at[pl.ds(wid * NT_ALL, NT_ALL)], valb)

    def _fire(g, b, coff):
        for k in range(4):
            pv = pckb[pl.ds(g * BE + 16 * k, 16)]
            gv = (pv & 0xFFFF) + coff
            pltpu.async_copy(x_hbm.at[gv], rbufs[b].at[pl.ds(16 * k, 16)],
                             sems[b])

    def _drain(b):
        pltpu.make_async_copy(x_hbm.at[pl.ds(0, BE)], rbufs[b],
                              sems[b]).wait()

    def _scale(g, b):
        rows = rbufs[b]

        def _sc16(j16, cc):
            vv = valb[pl.ds(g * BE + 16 * j16, 16)]
            for lane in range(16):
                s = vv[lane]
                r = j16 * 16 + lane
                for j in range(CW // 16):
                    sl = pl.ds(16 * j, 16)
                    rows[r, sl] = rows[r, sl] * s
            return cc
        lax.fori_loop(0, BE // 16, _sc16, 0)

    def _fire_s(g, b, hoff):
        for k in range(4):
            pv = pckb[pl.ds(g * BE + 16 * k, 16)]
            dv = lax.shift_right_logical(pv, 16) - hoff
            msk = (dv >= 0) & (dv < NH)
            dump = NH + ((lax.iota(_i32, 16) + 16 * k) & (NDUMP - 1))
            ivbufs[b][pl.ds(16 * k, 16)] = jnp.where(msk, dv, dump)
        pltpu.async_copy(rbufs[b], acc.at[ivbufs[b]], ssems[b], add=True)

    def _drain_s(b):
        pltpu.make_async_copy(rbufs[b], acc.at[ivbufs[b]], ssems[b]).wait()

    def _phase(p, carry):
        chunk = 2 * (p // 2) + cid
        coff = chunk * N
        hoff = (p % 2) * NH

        def _zr(i, c):
            for j in range(CW // 16):
                rows0[i, pl.ds(16 * j, 16)] = jnp.zeros((16,), _f32)
            return c
        lax.fori_loop(0, BE, _zr, 0)
        for i in range(7):
            pltpu.sync_copy(rows0, acc.at[pl.ds(wid * 504 + i * BE, BE)])
        pltpu.sync_copy(rows0.at[pl.ds(0, 56)],
                        acc.at[pl.ds(wid * 504 + 448, 56)])
        plsc.subcore_barrier()

        _fire(0, 0, coff)
        _drain(0)
        _fire(1, 1, coff)
        _scale(0, 0)
        _fire_s(0, 0, hoff)
        _drain(1)
        _scale(1, 1)
        _drain_s(0)
        _fire(2, 0, coff)
        _fire_s(1, 1, hoff)

        def _pair(g2, c):
            g = g2 * 2
            _drain_s(1)
            _fire(g + 1, 1, coff)
            _drain(0)
            _scale(g, 0)
            _fire_s(g, 0, hoff)
            _drain(1)
            _scale(g + 1, 1)
            _drain_s(0)
            _fire(g + 2, 0, coff)
            _fire_s(g + 1, 1, hoff)
            return c
        lax.fori_loop(1, NBB // 2 - 1, _pair, 0)
        _drain_s(1)
        _fire(NBB - 1, 1, coff)
        _drain(0)
        _scale(NBB - 2, 0)
        _fire_s(NBB - 2, 0, hoff)
        _drain(1)
        _scale(NBB - 1, 1)
        _drain_s(0)
        _fire_s(NBB - 1, 1, hoff)
        _drain_s(1)
        plsc.subcore_barrier()

        @pl.when(wid < 8)
        def _ro():
            for i in range(5):
                pltpu.sync_copy(
                    acc.at[pl.ds(wid * 1000 + i * 200, 200)],
                    out_hbm.at[pl.ds(coff + hoff + wid * 1000 + i * 200, 200)])
        plsc.subcore_barrier()
        return carry

    lax.fori_loop(0, 4, _phase, 0)


def _pad_idx(npad, salt):
    return ((jnp.arange(npad, dtype=_i32) * 131) + salt) % N


def kernel(x0, x1, x2, x3, x4,
           edge_index_0, edge_index_1, edge_index_2, edge_index_3,
           edge_value_0, edge_value_1, edge_value_2, edge_value_3,
           Wp0, Wp1, Wp2, Wp3, Wp4,
           bp0, bp1, bp2, bp3, bp4,
           alpha0, alpha1, W0, b0, W1, b1):
    xs = (x0, x1, x2, x3, x4)
    Wps = (Wp0, Wp1, Wp2, Wp3, Wp4)
    bps = (bp0, bp1, bp2, bp3, bp4)
    eidx = (edge_index_0, edge_index_1, edge_index_2, edge_index_3)
    evals = (edge_value_0, edge_value_1, edge_value_2, edge_value_3)

    beta_l0 = jnp.mean(jax.nn.softmax(alpha0, axis=-1), axis=0)
    beta_l1 = jnp.mean(jax.nn.softmax(alpha1, axis=-1), axis=0)
    betas16 = jnp.zeros((16,), _f32)
    betas16 = betas16.at[0:4].set(beta_l0[0:4]).at[8:12].set(beta_l1[0:4])

    X = jnp.concatenate(
        [_proj(x, w, b, bm=1000) for x, w, b in zip(xs, Wps, bps)], axis=0)
    x_cm = X.reshape(N, NCH, CW).transpose(1, 0, 2).reshape(NCH * N, CW)

    srcs, dsts, vals = [], [], []
    for t in range(4):
        npad = EPAD[t] - E_LIST[t]
        srcs += [eidx[t][0], _pad_idx(npad, 7 * t + 1)]
        dsts += [eidx[t][1], _pad_idx(npad, 13 * t + 3)]
        vals += [evals[t], jnp.zeros((npad,), _f32)]
    self_idx = jnp.arange(N, dtype=_i32)
    tail = E_ALL - EP4 - N
    srcs += [self_idx, _pad_idx(tail, 5)]
    dsts += [self_idx, _pad_idx(tail, 9)]
    vals += [jnp.zeros((N + tail,), _f32)]
    src_all = jnp.concatenate(srcs)
    dst_all = jnp.concatenate(dsts)
    val_all = jnp.concatenate(vals)
    pck_all = src_all | (dst_all << 16)

    sv0_p, sv1_p, _ = _prep_sc(dst_all, val_all, betas16)
    sval0 = jnp.concatenate(
        [sv0_p, jnp.full((N,), beta_l0[4], _f32), jnp.zeros((tail,), _f32)])
    sval1 = jnp.concatenate(
        [sv1_p, jnp.full((N,), beta_l1[4], _f32), jnp.zeros((tail,), _f32)])

    h1 = _spmm_sc(pck_all, sval0, x_cm)
    x2_cm = _layer_tc(h1, W0, b0, chunked_out=True)

    h2 = _spmm_sc(pck_all, sval1, x2_cm)
    return _layer_tc(h2, W1, b1, chunked_out=False)


# speedup vs baseline: 4.1987x; 1.0617x over previous
import functools

import jax
import jax.numpy as jnp
from jax import lax
from jax.experimental import pallas as pl
from jax.experimental.pallas import tpu as pltpu
from jax.experimental.pallas import tpu_sc as plsc

N = 16000
D = 512
NCH = 4
CW = 128
NPH = NCH // 2
E_LIST = (100000, 100000, 100000, 32000)
EPAD = (100352, 100352, 100352, 32768)
TYPE_BASE = (0, 100352, 200704, 301056)
EP4 = 333824
E_ALL = 350720
NT_ALL = E_ALL // 16

_f32 = jnp.float32
_i32 = jnp.int32


def _proj_body(x_ref, w_ref, b_ref, o_ref):
    acc = jnp.dot(x_ref[...], w_ref[...], preferred_element_type=_f32)
    o_ref[...] = acc + b_ref[...][None, :]


def _proj(x, w, b, bm):
    m, k = x.shape
    return pl.pallas_call(
        _proj_body,
        grid=(m // bm,),
        in_specs=[
            pl.BlockSpec((bm, k), lambda i: (i, 0)),
            pl.BlockSpec((k, D), lambda i: (0, 0)),
            pl.BlockSpec((D,), lambda i: (0,)),
        ],
        out_specs=pl.BlockSpec((bm, D), lambda i: (i, 0)),
        out_shape=jax.ShapeDtypeStruct((m, D), _f32),
    )(x, w, b)


def _layer_body_chunked(h_ref, w_ref, b_ref, o_ref):
    acc = jnp.dot(h_ref[0], w_ref[0], preferred_element_type=_f32)
    for c in range(1, NCH):
        acc += jnp.dot(h_ref[c], w_ref[c], preferred_element_type=_f32)
    acc = jnp.maximum(acc + b_ref[...][None, :], 0.0)
    for c in range(NCH):
        o_ref[c] = acc[:, c * CW:(c + 1) * CW]


def _layer_body_flat(h_ref, w_ref, b_ref, o_ref):
    acc = jnp.dot(h_ref[0], w_ref[0], preferred_element_type=_f32)
    for c in range(1, NCH):
        acc += jnp.dot(h_ref[c], w_ref[c], preferred_element_type=_f32)
    o_ref[...] = jnp.maximum(acc + b_ref[...][None, :], 0.0)


def _layer_tc(h_cm, w, b, chunked_out, bm=1000):
    h4 = h_cm.reshape(NCH, N, CW)
    w4 = w.reshape(NCH, CW, D)
    in_specs = [
        pl.BlockSpec((NCH, bm, CW), lambda i: (0, i, 0)),
        pl.BlockSpec((NCH, CW, D), lambda i: (0, 0, 0)),
        pl.BlockSpec((D,), lambda i: (0,)),
    ]
    if chunked_out:
        out = pl.pallas_call(
            _layer_body_chunked,
            grid=(N // bm,),
            in_specs=in_specs,
            out_specs=pl.BlockSpec((NCH, bm, CW), lambda i: (0, i, 0)),
            out_shape=jax.ShapeDtypeStruct((NCH, N, CW), _f32),
        )(h4, w4, b)
        return out.reshape(NCH * N, CW)
    return pl.pallas_call(
        _layer_body_flat,
        grid=(N // bm,),
        in_specs=in_specs,
        out_specs=pl.BlockSpec((bm, D), lambda i: (i, 0)),
        out_shape=jax.ShapeDtypeStruct((N, D), _f32),
    )(h4, w4, b)


_MESH = plsc.VectorSubcoreMesh(core_axis_name="c", subcore_axis_name="s")


@functools.partial(
    pl.kernel,
    out_type=(
        jax.ShapeDtypeStruct((EP4,), _f32),
        jax.ShapeDtypeStruct((EP4,), _f32),
        jax.ShapeDtypeStruct((2 * N,), _f32),
    ),
    mesh=_MESH,
    scratch_types=[
        pltpu.VMEM_SHARED((N,), _f32),
        pltpu.VMEM((1008,), _f32),
        pltpu.VMEM((128,), _i32),
        pltpu.VMEM((128,), _f32),
        pltpu.VMEM((128,), _f32),
        pltpu.VMEM((128,), _f32),
        pltpu.VMEM((128,), _f32),
        pltpu.VMEM((1024,), _f32),
        pltpu.VMEM((16,), _f32),
        pltpu.SemaphoreType.DMA,
    ],
)
def _prep_sc(dst_hbm, val_hbm, betas_hbm, sv0_hbm, sv1_hbm, dinv_hbm,
             deg_sh, dslice, dstb, valb, dvb, o0, o1, zbuf, btile, sem):
    cid = lax.axis_index("c")
    wid = lax.axis_index("s")

    def _zb(i, c):
        zbuf[pl.ds(16 * i, 16)] = jnp.zeros((16,), _f32)
        return c
    lax.fori_loop(0, 64, _zb, 0)
    pltpu.sync_copy(betas_hbm, btile)

    for t in range(4):
        nt = EPAD[t] // 16
        nb = nt // 128
        base = TYPE_BASE[t]

        @pl.when(cid == (t % 2))
        def _type_block():
            pltpu.sync_copy(zbuf.at[pl.ds(0, 1000)],
                            deg_sh.at[pl.ds(wid * 1000, 1000)])
            plsc.subcore_barrier()

            def _deg(g, c):
                off = base + wid * nt + g * 128
                pltpu.sync_copy(dst_hbm.at[pl.ds(off, 128)], dstb)
                pltpu.sync_copy(val_hbm.at[pl.ds(off, 128)], valb)
                pltpu.sync_copy(valb, deg_sh.at[dstb], add=True)
                return c
            lax.fori_loop(0, nb, _deg, 0)
            plsc.subcore_barrier()

            pltpu.sync_copy(deg_sh.at[pl.ds(wid * 1000, 1000)],
                            dslice.at[pl.ds(0, 1000)])

            def _inv(i, c):
                v = dslice[pl.ds(16 * i, 16)]
                pos = v > 0.0
                dslice[pl.ds(16 * i, 16)] = jnp.where(
                    pos, 1.0 / jnp.where(pos, v, 1.0), 0.0)
                return c
            lax.fori_loop(0, 63, _inv, 0)
            pltpu.sync_copy(dslice.at[pl.ds(0, 1000)],
                            dinv_hbm.at[pl.ds(cid * N + wid * 1000, 1000)])
            plsc.subcore_barrier()

            bvec = btile[pl.ds(0, 16)]
            b0s = bvec[t]
            b1s = bvec[8 + t]
            tab_off = cid * N

            def _sval(g, c):
                off = base + wid * nt + g * 128
                pltpu.sync_copy(dst_hbm.at[pl.ds(off, 128)], dstb)
                pltpu.sync_copy(val_hbm.at[pl.ds(off, 128)], valb)
                for j in range(8):
                    sl = pl.ds(16 * j, 16)
                    dstb[sl] = dstb[sl] + tab_off
                pltpu.async_copy(dinv_hbm.at[dstb], dvb, sem).wait()
                for j in range(8):
                    sl = pl.ds(16 * j, 16)
                    nv = valb[sl] * dvb[sl]
                    o0[sl] = nv * b0s
                    o1[sl] = nv * b1s
                pltpu.sync_copy(o0, sv0_hbm.at[pl.ds(off, 128)])
                pltpu.sync_copy(o1, sv1_hbm.at[pl.ds(off, 128)])
                return c
            lax.fori_loop(0, nb, _sval, 0)


NH = 8000
NDUMP = 64
BE = 80
NBB = NT_ALL // BE


@functools.partial(
    pl.kernel,
    out_type=jax.ShapeDtypeStruct((NCH * N, CW), _f32),
    mesh=_MESH,
    scratch_types=[
        pltpu.VMEM_SHARED((NH + NDUMP, CW), _f32),
        pltpu.VMEM((NT_ALL,), _i32),
        pltpu.VMEM((NT_ALL,), _f32),
        pltpu.VMEM((BE, CW), _f32),
        pltpu.VMEM((BE, CW), _f32),
        pltpu.VMEM((BE,), _i32),
        pltpu.VMEM((BE,), _i32),
        pltpu.SemaphoreType.DMA,
        pltpu.SemaphoreType.DMA,
        pltpu.SemaphoreType.DMA,
        pltpu.SemaphoreType.DMA,
    ],
)
def _spmm_sc(pck_hbm, sval_hbm, x_hbm, out_hbm,
             acc, pckb, valb, rows0, rows1, iv0, iv1,
             sem0, sem1, ssem0, ssem1):
    cid = lax.axis_index("c")
    wid = lax.axis_index("s")
    rbufs = (rows0, rows1)
    ivbufs = (iv0, iv1)
    sems = (sem0, sem1)
    ssems = (ssem0, ssem1)

    pltpu.sync_copy(pck_hbm.at[pl.ds(wid * NT_ALL, NT_ALL)], pckb)
    pltpu.sync_copy(sval_hbm.at[pl.ds(wid * NT_ALL, NT_ALL)], valb)

    def _fire(g, b, coff):
        for k in range(BE // 16):
            pv = pckb[pl.ds(g * BE + 16 * k, 16)]
            gv = (pv & 0xFFFF) + coff
            pltpu.async_copy(x_hbm.at[gv], rbufs[b].at[pl.ds(16 * k, 16)],
                             sems[b])

    def _drain(b):
        pltpu.make_async_copy(x_hbm.at[pl.ds(0, BE)], rbufs[b],
                              sems[b]).wait()

    def _scale(g, b):
        rows = rbufs[b]

        def _sc16(j16, cc):
            vv = valb[pl.ds(g * BE + 16 * j16, 16)]
            for lane in range(16):
                s = vv[lane]
                r = j16 * 16 + lane
                for j in range(CW // 16):
                    sl = pl.ds(16 * j, 16)
                    rows[r, sl] = rows[r, sl] * s
            return cc
        lax.fori_loop(0, BE // 16, _sc16, 0)

    def _fire_s(g, b, hoff):
        for k in range(BE // 16):
            pv = pckb[pl.ds(g * BE + 16 * k, 16)]
            dv = lax.shift_right_logical(pv, 16) - hoff
            msk = (dv >= 0) & (dv < NH)
            dump = NH + ((lax.iota(_i32, 16) + 16 * k) & (NDUMP - 1))
            ivbufs[b][pl.ds(16 * k, 16)] = jnp.where(msk, dv, dump)
        pltpu.async_copy(rbufs[b], acc.at[ivbufs[b]], ssems[b], add=True)

    def _drain_s(b):
        pltpu.make_async_copy(rbufs[b], acc.at[ivbufs[b]], ssems[b]).wait()

    def _phase(p, carry):
        chunk = 2 * (p // 2) + cid
        coff = chunk * N
        hoff = (p % 2) * NH

        def _zr(i, c):
            for j in range(CW // 16):
                rows0[i, pl.ds(16 * j, 16)] = jnp.zeros((16,), _f32)
            return c
        lax.fori_loop(0, BE, _zr, 0)
        for i in range(6):
            pltpu.sync_copy(rows0, acc.at[pl.ds(wid * 504 + i * BE, BE)])
        pltpu.sync_copy(rows0.at[pl.ds(0, 24)],
                        acc.at[pl.ds(wid * 504 + 480, 24)])
        plsc.subcore_barrier()

        _fire(0, 0, coff)
        _drain(0)
        _fire(1, 1, coff)
        _scale(0, 0)
        _fire_s(0, 0, hoff)
        _drain(1)
        _scale(1, 1)
        _drain_s(0)
        _fire(2, 0, coff)
        _fire_s(1, 1, hoff)

        def _pair(g2, c):
            g = g2 * 2
            _drain_s(1)
            _fire(g + 1, 1, coff)
            _drain(0)
            _scale(g, 0)
            _fire_s(g, 0, hoff)
            _drain(1)
            _scale(g + 1, 1)
            _drain_s(0)
            _fire(g + 2, 0, coff)
            _fire_s(g + 1, 1, hoff)
            return c
        lax.fori_loop(1, NBB // 2 - 1, _pair, 0)
        _drain_s(1)
        _fire(NBB - 1, 1, coff)
        _drain(0)
        _scale(NBB - 2, 0)
        _fire_s(NBB - 2, 0, hoff)
        _drain(1)
        _scale(NBB - 1, 1)
        _drain_s(0)
        _fire_s(NBB - 1, 1, hoff)
        _drain_s(1)
        plsc.subcore_barrier()

        @pl.when(wid < 8)
        def _ro():
            for i in range(5):
                pltpu.sync_copy(
                    acc.at[pl.ds(wid * 1000 + i * 200, 200)],
                    out_hbm.at[pl.ds(coff + hoff + wid * 1000 + i * 200, 200)])
        plsc.subcore_barrier()
        return carry

    lax.fori_loop(0, 4, _phase, 0)


def _pad_idx(npad, salt):
    return ((jnp.arange(npad, dtype=_i32) * 131) + salt) % N


def kernel(x0, x1, x2, x3, x4,
           edge_index_0, edge_index_1, edge_index_2, edge_index_3,
           edge_value_0, edge_value_1, edge_value_2, edge_value_3,
           Wp0, Wp1, Wp2, Wp3, Wp4,
           bp0, bp1, bp2, bp3, bp4,
           alpha0, alpha1, W0, b0, W1, b1):
    xs = (x0, x1, x2, x3, x4)
    Wps = (Wp0, Wp1, Wp2, Wp3, Wp4)
    bps = (bp0, bp1, bp2, bp3, bp4)
    eidx = (edge_index_0, edge_index_1, edge_index_2, edge_index_3)
    evals = (edge_value_0, edge_value_1, edge_value_2, edge_value_3)

    beta_l0 = jnp.mean(jax.nn.softmax(alpha0, axis=-1), axis=0)
    beta_l1 = jnp.mean(jax.nn.softmax(alpha1, axis=-1), axis=0)
    betas16 = jnp.zeros((16,), _f32)
    betas16 = betas16.at[0:4].set(beta_l0[0:4]).at[8:12].set(beta_l1[0:4])

    X = jnp.concatenate(
        [_proj(x, w, b, bm=1000) for x, w, b in zip(xs, Wps, bps)], axis=0)
    x_cm = X.reshape(N, NCH, CW).transpose(1, 0, 2).reshape(NCH * N, CW)

    srcs, dsts, vals = [], [], []
    for t in range(4):
        npad = EPAD[t] - E_LIST[t]
        srcs += [eidx[t][0], _pad_idx(npad, 7 * t + 1)]
        dsts += [eidx[t][1], _pad_idx(npad, 13 * t + 3)]
        vals += [evals[t], jnp.zeros((npad,), _f32)]
    self_idx = jnp.arange(N, dtype=_i32)
    tail = E_ALL - EP4 - N
    srcs += [self_idx, _pad_idx(tail, 5)]
    dsts += [self_idx, _pad_idx(tail, 9)]
    vals += [jnp.zeros((N + tail,), _f32)]
    src_all = jnp.concatenate(srcs)
    dst_all = jnp.concatenate(dsts)
    val_all = jnp.concatenate(vals)
    pck_all = src_all | (dst_all << 16)

    sv0_p, sv1_p, _ = _prep_sc(dst_all, val_all, betas16)
    sval0 = jnp.concatenate(
        [sv0_p, jnp.full((N,), beta_l0[4], _f32), jnp.zeros((tail,), _f32)])
    sval1 = jnp.concatenate(
        [sv1_p, jnp.full((N,), beta_l1[4], _f32), jnp.zeros((tail,), _f32)])

    h1 = _spmm_sc(pck_all, sval0, x_cm)
    x2_cm = _layer_tc(h1, W0, b0, chunked_out=True)

    h2 = _spmm_sc(pck_all, sval1, x2_cm)
    return _layer_tc(h2, W1, b1, chunked_out=False)


# speedup vs baseline: 4.4470x; 1.0591x over previous
import functools

import jax
import jax.numpy as jnp
from jax import lax
from jax.experimental import pallas as pl
from jax.experimental.pallas import tpu as pltpu
from jax.experimental.pallas import tpu_sc as plsc

N = 16000
D = 512
NCH = 4
CW = 128
NPH = NCH // 2
E_LIST = (100000, 100000, 100000, 32000)
EPAD = (100352, 100352, 100352, 32768)
TYPE_BASE = (0, 100352, 200704, 301056)
EP4 = 333824
E_ALL = 350720
NT_ALL = E_ALL // 16

_f32 = jnp.float32
_i32 = jnp.int32


def _proj_body(x_ref, w_ref, b_ref, o_ref):
    acc = jnp.dot(x_ref[...], w_ref[...], preferred_element_type=_f32)
    o_ref[...] = acc + b_ref[...][None, :]


def _proj(x, w, b, bm):
    m, k = x.shape
    return pl.pallas_call(
        _proj_body,
        grid=(m // bm,),
        in_specs=[
            pl.BlockSpec((bm, k), lambda i: (i, 0)),
            pl.BlockSpec((k, D), lambda i: (0, 0)),
            pl.BlockSpec((D,), lambda i: (0,)),
        ],
        out_specs=pl.BlockSpec((bm, D), lambda i: (i, 0)),
        out_shape=jax.ShapeDtypeStruct((m, D), _f32),
    )(x, w, b)


def _layer_body_chunked(h_ref, w_ref, b_ref, o_ref):
    acc = jnp.dot(h_ref[0], w_ref[0], preferred_element_type=_f32)
    for c in range(1, NCH):
        acc += jnp.dot(h_ref[c], w_ref[c], preferred_element_type=_f32)
    acc = jnp.maximum(acc + b_ref[...][None, :], 0.0)
    for c in range(NCH):
        o_ref[c] = acc[:, c * CW:(c + 1) * CW]


def _layer_body_flat(h_ref, w_ref, b_ref, o_ref):
    acc = jnp.dot(h_ref[0], w_ref[0], preferred_element_type=_f32)
    for c in range(1, NCH):
        acc += jnp.dot(h_ref[c], w_ref[c], preferred_element_type=_f32)
    o_ref[...] = jnp.maximum(acc + b_ref[...][None, :], 0.0)


def _layer_tc(h_cm, w, b, chunked_out, bm=1000):
    h4 = h_cm.reshape(NCH, N, CW)
    w4 = w.reshape(NCH, CW, D)
    in_specs = [
        pl.BlockSpec((NCH, bm, CW), lambda i: (0, i, 0)),
        pl.BlockSpec((NCH, CW, D), lambda i: (0, 0, 0)),
        pl.BlockSpec((D,), lambda i: (0,)),
    ]
    if chunked_out:
        out = pl.pallas_call(
            _layer_body_chunked,
            grid=(N // bm,),
            in_specs=in_specs,
            out_specs=pl.BlockSpec((NCH, bm, CW), lambda i: (0, i, 0)),
            out_shape=jax.ShapeDtypeStruct((NCH, N, CW), _f32),
        )(h4, w4, b)
        return out.reshape(NCH * N, CW)
    return pl.pallas_call(
        _layer_body_flat,
        grid=(N // bm,),
        in_specs=in_specs,
        out_specs=pl.BlockSpec((bm, D), lambda i: (i, 0)),
        out_shape=jax.ShapeDtypeStruct((N, D), _f32),
    )(h4, w4, b)


_MESH = plsc.VectorSubcoreMesh(core_axis_name="c", subcore_axis_name="s")


@functools.partial(
    pl.kernel,
    out_type=(
        jax.ShapeDtypeStruct((EP4,), _f32),
        jax.ShapeDtypeStruct((EP4,), _f32),
        jax.ShapeDtypeStruct((2 * N,), _f32),
    ),
    mesh=_MESH,
    scratch_types=[
        pltpu.VMEM_SHARED((N,), _f32),
        pltpu.VMEM((6400,), _i32),
        pltpu.VMEM((6400,), _f32),
        pltpu.VMEM((1008,), _f32),
        pltpu.VMEM((128,), _i32),
        pltpu.VMEM((128,), _f32),
        pltpu.VMEM((128,), _f32),
        pltpu.VMEM((128,), _f32),
        pltpu.VMEM((128,), _f32),
        pltpu.VMEM((1024,), _f32),
        pltpu.VMEM((16,), _f32),
        pltpu.SemaphoreType.DMA,
        pltpu.SemaphoreType.DMA,
    ],
)
def _prep_sc(dst_hbm, val_hbm, betas_hbm, sv0_hbm, sv1_hbm, dinv_hbm,
             deg_sh, dstblk, valblk, dslice, dstb, dvb0, dvb1, o0, o1,
             zbuf, btile, gsem0, gsem1):
    cid = lax.axis_index("c")
    wid = lax.axis_index("s")
    dvbs = (dvb0, dvb1)
    gsems = (gsem0, gsem1)

    def _zb(i, c):
        zbuf[pl.ds(16 * i, 16)] = jnp.zeros((16,), _f32)
        return c
    lax.fori_loop(0, 64, _zb, 0)
    pltpu.sync_copy(betas_hbm, btile)

    for t in range(4):
        nt = EPAD[t] // 16
        nb = nt // 128
        base = TYPE_BASE[t]

        @pl.when(cid == (t % 2))
        def _type_block():
            pltpu.sync_copy(dst_hbm.at[pl.ds(base + wid * nt, nt)],
                            dstblk.at[pl.ds(0, nt)])
            pltpu.sync_copy(val_hbm.at[pl.ds(base + wid * nt, nt)],
                            valblk.at[pl.ds(0, nt)])
            pltpu.sync_copy(zbuf.at[pl.ds(0, 1000)],
                            deg_sh.at[pl.ds(wid * 1000, 1000)])
            plsc.subcore_barrier()

            def _deg(g, c):
                for j in range(8):
                    sl = pl.ds(16 * j, 16)
                    dstb[sl] = dstblk[pl.ds(g * 128 + 16 * j, 16)]
                pltpu.sync_copy(valblk.at[pl.ds(g * 128, 128)],
                                deg_sh.at[dstb], add=True)
                return c
            lax.fori_loop(0, nb, _deg, 0)
            plsc.subcore_barrier()

            pltpu.sync_copy(deg_sh.at[pl.ds(wid * 1000, 1000)],
                            dslice.at[pl.ds(0, 1000)])

            def _inv(i, c):
                v = dslice[pl.ds(16 * i, 16)]
                pos = v > 0.0
                dslice[pl.ds(16 * i, 16)] = jnp.where(
                    pos, 1.0 / jnp.where(pos, v, 1.0), 0.0)
                return c
            lax.fori_loop(0, 63, _inv, 0)
            pltpu.sync_copy(dslice.at[pl.ds(0, 1000)],
                            dinv_hbm.at[pl.ds(cid * N + wid * 1000, 1000)])
            plsc.subcore_barrier()

            def _sh(i, c):
                sl = pl.ds(16 * i, 16)
                dstblk[sl] = dstblk[sl] + cid * N
                return c
            lax.fori_loop(0, nt // 16, _sh, 0)

            bvec = btile[pl.ds(0, 16)]
            b0s = bvec[t]
            b1s = bvec[8 + t]

            def _fire_g(g, b):
                pltpu.async_copy(
                    dinv_hbm.at[dstblk.at[pl.ds(g * 128, 128)]],
                    dvbs[b], gsems[b])

            def _drain_g(g, b):
                pltpu.make_async_copy(
                    dinv_hbm.at[dstblk.at[pl.ds(g * 128, 128)]],
                    dvbs[b], gsems[b]).wait()

            def _emit(g, b):
                off = base + wid * nt + g * 128
                for j in range(8):
                    sl = pl.ds(16 * j, 16)
                    nv = valblk[pl.ds(g * 128 + 16 * j, 16)] * dvbs[b][sl]
                    o0[sl] = nv * b0s
                    o1[sl] = nv * b1s
                pltpu.sync_copy(o0, sv0_hbm.at[pl.ds(off, 128)])
                pltpu.sync_copy(o1, sv1_hbm.at[pl.ds(off, 128)])

            _fire_g(0, 0)

            def _gpair(k, c):
                g = 2 * k
                _drain_g(g, 0)
                _fire_g(g + 1, 1)
                _emit(g, 0)
                _drain_g(g + 1, 1)
                _fire_g(g + 2, 0)
                _emit(g + 1, 1)
                return c
            lax.fori_loop(0, (nb - 1) // 2, _gpair, 0)
            if nb % 2 == 1:
                _drain_g(nb - 1, 0)
                _emit(nb - 1, 0)
            else:
                _drain_g(nb - 2, 0)
                _fire_g(nb - 1, 1)
                _emit(nb - 2, 0)
                _drain_g(nb - 1, 1)
                _emit(nb - 1, 1)


NH = 8000
NDUMP = 64
BE = 80
NBB = NT_ALL // BE


@functools.partial(
    pl.kernel,
    out_type=jax.ShapeDtypeStruct((NCH * N, CW), _f32),
    mesh=_MESH,
    scratch_types=[
        pltpu.VMEM_SHARED((NH + NDUMP, CW), _f32),
        pltpu.VMEM((NT_ALL,), _i32),
        pltpu.VMEM((NT_ALL,), _f32),
        pltpu.VMEM((BE, CW), _f32),
        pltpu.VMEM((BE, CW), _f32),
        pltpu.VMEM((BE,), _i32),
        pltpu.VMEM((BE,), _i32),
        pltpu.SemaphoreType.DMA,
        pltpu.SemaphoreType.DMA,
        pltpu.SemaphoreType.DMA,
        pltpu.SemaphoreType.DMA,
    ],
)
def _spmm_sc(pck_hbm, sval_hbm, x_hbm, out_hbm,
             acc, pckb, valb, rows0, rows1, iv0, iv1,
             sem0, sem1, ssem0, ssem1):
    cid = lax.axis_index("c")
    wid = lax.axis_index("s")
    rbufs = (rows0, rows1)
    ivbufs = (iv0, iv1)
    sems = (sem0, sem1)
    ssems = (ssem0, ssem1)

    pltpu.sync_copy(pck_hbm.at[pl.ds(wid * NT_ALL, NT_ALL)], pckb)
    pltpu.sync_copy(sval_hbm.at[pl.ds(wid * NT_ALL, NT_ALL)], valb)

    def _fire(g, b, coff):
        for k in range(BE // 16):
            pv = pckb[pl.ds(g * BE + 16 * k, 16)]
            gv = (pv & 0xFFFF) + coff
            pltpu.async_copy(x_hbm.at[gv], rbufs[b].at[pl.ds(16 * k, 16)],
                             sems[b])

    def _drain(b):
        pltpu.make_async_copy(x_hbm.at[pl.ds(0, BE)], rbufs[b],
                              sems[b]).wait()

    def _scale(g, b):
        rows = rbufs[b]

        def _sc16(j16, cc):
            vv = valb[pl.ds(g * BE + 16 * j16, 16)]
            for lane in range(16):
                s = vv[lane]
                r = j16 * 16 + lane
                for j in range(CW // 16):
                    sl = pl.ds(16 * j, 16)
                    rows[r, sl] = rows[r, sl] * s
            return cc
        lax.fori_loop(0, BE // 16, _sc16, 0)

    def _fire_s(g, b, hoff):
        for k in range(BE // 16):
            pv = pckb[pl.ds(g * BE + 16 * k, 16)]
            dv = lax.shift_right_logical(pv, 16) - hoff
            msk = (dv >= 0) & (dv < NH)
            dump = NH + ((lax.iota(_i32, 16) + 16 * k) & (NDUMP - 1))
            ivbufs[b][pl.ds(16 * k, 16)] = jnp.where(msk, dv, dump)
        pltpu.async_copy(rbufs[b], acc.at[ivbufs[b]], ssems[b], add=True)

    def _drain_s(b):
        pltpu.make_async_copy(rbufs[b], acc.at[ivbufs[b]], ssems[b]).wait()

    def _phase(p, carry):
        chunk = 2 * (p // 2) + cid
        coff = chunk * N
        hoff = (p % 2) * NH

        def _zr(i, c):
            for j in range(CW // 16):
                rows0[i, pl.ds(16 * j, 16)] = jnp.zeros((16,), _f32)
            return c
        lax.fori_loop(0, BE, _zr, 0)
        for i in range(6):
            pltpu.sync_copy(rows0, acc.at[pl.ds(wid * 504 + i * BE, BE)])
        pltpu.sync_copy(rows0.at[pl.ds(0, 24)],
                        acc.at[pl.ds(wid * 504 + 480, 24)])
        plsc.subcore_barrier()

        _fire(0, 0, coff)
        _drain(0)
        _fire(1, 1, coff)
        _scale(0, 0)
        _fire_s(0, 0, hoff)
        _drain(1)
        _scale(1, 1)
        _drain_s(0)
        _fire(2, 0, coff)
        _fire_s(1, 1, hoff)

        def _pair(g2, c):
            g = g2 * 2
            _drain_s(1)
            _fire(g + 1, 1, coff)
            _drain(0)
            _scale(g, 0)
            _fire_s(g, 0, hoff)
            _drain(1)
            _scale(g + 1, 1)
            _drain_s(0)
            _fire(g + 2, 0, coff)
            _fire_s(g + 1, 1, hoff)
            return c
        lax.fori_loop(1, NBB // 2 - 1, _pair, 0)
        _drain_s(1)
        _fire(NBB - 1, 1, coff)
        _drain(0)
        _scale(NBB - 2, 0)
        _fire_s(NBB - 2, 0, hoff)
        _drain(1)
        _scale(NBB - 1, 1)
        _drain_s(0)
        _fire_s(NBB - 1, 1, hoff)
        _drain_s(1)
        plsc.subcore_barrier()

        @pl.when(wid < 8)
        def _ro():
            for i in range(5):
                pltpu.sync_copy(
                    acc.at[pl.ds(wid * 1000 + i * 200, 200)],
                    out_hbm.at[pl.ds(coff + hoff + wid * 1000 + i * 200, 200)])
        plsc.subcore_barrier()
        return carry

    lax.fori_loop(0, 4, _phase, 0)


def _pad_idx(npad, salt):
    return ((jnp.arange(npad, dtype=_i32) * 131) + salt) % N


def kernel(x0, x1, x2, x3, x4,
           edge_index_0, edge_index_1, edge_index_2, edge_index_3,
           edge_value_0, edge_value_1, edge_value_2, edge_value_3,
           Wp0, Wp1, Wp2, Wp3, Wp4,
           bp0, bp1, bp2, bp3, bp4,
           alpha0, alpha1, W0, b0, W1, b1):
    xs = (x0, x1, x2, x3, x4)
    Wps = (Wp0, Wp1, Wp2, Wp3, Wp4)
    bps = (bp0, bp1, bp2, bp3, bp4)
    eidx = (edge_index_0, edge_index_1, edge_index_2, edge_index_3)
    evals = (edge_value_0, edge_value_1, edge_value_2, edge_value_3)

    beta_l0 = jnp.mean(jax.nn.softmax(alpha0, axis=-1), axis=0)
    beta_l1 = jnp.mean(jax.nn.softmax(alpha1, axis=-1), axis=0)
    betas16 = jnp.zeros((16,), _f32)
    betas16 = betas16.at[0:4].set(beta_l0[0:4]).at[8:12].set(beta_l1[0:4])

    X = jnp.concatenate(
        [_proj(x, w, b, bm=1000) for x, w, b in zip(xs, Wps, bps)], axis=0)
    x_cm = X.reshape(N, NCH, CW).transpose(1, 0, 2).reshape(NCH * N, CW)

    srcs, dsts, vals = [], [], []
    for t in range(4):
        npad = EPAD[t] - E_LIST[t]
        srcs += [eidx[t][0], _pad_idx(npad, 7 * t + 1)]
        dsts += [eidx[t][1], _pad_idx(npad, 13 * t + 3)]
        vals += [evals[t], jnp.zeros((npad,), _f32)]
    self_idx = jnp.arange(N, dtype=_i32)
    tail = E_ALL - EP4 - N
    srcs += [self_idx, _pad_idx(tail, 5)]
    dsts += [self_idx, _pad_idx(tail, 9)]
    vals += [jnp.zeros((N + tail,), _f32)]
    src_all = jnp.concatenate(srcs)
    dst_all = jnp.concatenate(dsts)
    val_all = jnp.concatenate(vals)
    pck_all = src_all | (dst_all << 16)

    sv0_p, sv1_p, _ = _prep_sc(dst_all, val_all, betas16)
    sval0 = jnp.concatenate(
        [sv0_p, jnp.full((N,), beta_l0[4], _f32), jnp.zeros((tail,), _f32)])
    sval1 = jnp.concatenate(
        [sv1_p, jnp.full((N,), beta_l1[4], _f32), jnp.zeros((tail,), _f32)])

    h1 = _spmm_sc(pck_all, sval0, x_cm)
    x2_cm = _layer_tc(h1, W0, b0, chunked_out=True)

    h2 = _spmm_sc(pck_all, sval1, x2_cm)
    return _layer_tc(h2, W1, b1, chunked_out=False)
